# baseline (speedup 1.0000x reference)
import functools

import jax
import jax.numpy as jnp
from jax import lax
from jax.experimental import pallas as pl
from jax.experimental.pallas import tpu as pltpu
from jax.experimental.pallas import tpu_sc as plsc

N = 10000
NP = 10240
E = 320000
NW = 32
ET = 10240
EP = NW * ET
G = 64
NSUP = 20
GSUP = 8
NG = NSUP * GSUP
D = 128
NUM_GRAPHS = 64
STRIPE = NP // 16
NT = 10112
SCNC = 2


def _tc_pro_kernel(x_ref, W_ref, asrc_ref, adst_ref, h_ref, as_ref, ad_ref):
    h = jnp.dot(x_ref[...], W_ref[...], preferred_element_type=jnp.float32)
    h_ref[...] = h
    as_ref[...] = jnp.dot(h, asrc_ref[...])
    ad_ref[...] = jnp.dot(h, adst_ref[...])


def _combine(nump_ref, denp_ref, h_ref, as_ref, ad_ref, b_ref):
    h = h_ref[...]
    al = as_ref[...] + ad_ref[...]
    wl = jnp.exp(jnp.where(al >= 0, al, 0.2 * al))
    num = nump_ref[0] + nump_ref[1] + wl[:, None] * h
    den = jnp.sum(denp_ref[...].reshape(NW, NP), axis=0) + wl
    return num / den[:, None] + b_ref[...]


def _tc_mid_kernel(nump_ref, denp_ref, h_ref, as_ref, ad_ref, b_ref, W_ref,
                   asrc_ref, adst_ref, h2_ref, as2_ref, ad2_ref):
    h1 = jnp.maximum(_combine(nump_ref, denp_ref, h_ref, as_ref, ad_ref, b_ref), 0.0)
    h2 = jnp.dot(h1, W_ref[...], preferred_element_type=jnp.float32)
    h2_ref[...] = h2
    as2_ref[...] = jnp.dot(h2, asrc_ref[...])
    ad2_ref[...] = jnp.dot(h2, adst_ref[...])


def _tc_fin_kernel(nump_ref, denp_ref, h_ref, as_ref, ad_ref, b_ref, batch_ref,
                   out_ref):
    hf = _combine(nump_ref, denp_ref, h_ref, as_ref, ad_ref, b_ref)[:N]
    bat = batch_ref[...]
    onehot = (bat[:, None] == lax.broadcasted_iota(jnp.int32, (N, NUM_GRAPHS), 1)
              ).astype(jnp.float32)
    s = lax.dot_general(onehot, hf, (((0,), (0,)), ((), ())),
                        preferred_element_type=jnp.float32)
    cnt = jnp.sum(onehot, axis=0)
    out_ref[...] = s / jnp.maximum(cnt, 1.0)[:, None]


def _zero_rows(buf):
    zero16 = jnp.zeros((16,), jnp.float32)

    @pl.loop(0, G)
    def _(r):
        for k in range(D // 16):
            buf[r, pl.ds(k * 16, 16)] = zero16


def _sc_edge_kernel(hp, asn, adn, srcg, dstg, num_out, den_out,
                    src_v, dst_v, as_v, ad_v, wA, wB, den_v, bufA, bufB, num_sh,
                    semGA, semGB, semSA, semSB):
    c = lax.axis_index("c")
    s = lax.axis_index("s")
    wid = s * SCNC + c
    base = s * STRIPE

    pltpu.sync_copy(asn.at[pl.ds(0, NT)], as_v)
    pltpu.sync_copy(adn.at[pl.ds(0, NT)], ad_v)

    zero16 = jnp.zeros((16,), jnp.float32)

    @pl.loop(0, NT // 16)
    def _(i):
        den_v[pl.ds(i * 16, 16)] = zero16

    _zero_rows(bufA)
    _zero_rows(bufB)

    for j in range(STRIPE // G):
        pltpu.sync_copy(bufA, num_sh.at[pl.ds(base + j * G, G)])

    plsc.subcore_barrier()

    def compute_w(j, w_v):
        for k in range(G // 16):
            src16 = src_v[j, pl.ds(k * 16, 16)]
            dst16 = dst_v[j, pl.ds(k * 16, 16)]
            e16 = (plsc.load_gather(as_v, [src16])
                   + plsc.load_gather(ad_v, [dst16]))
            e16 = jnp.where(e16 >= 0, e16, 0.2 * e16)
            w16 = jnp.exp(e16)
            w_v[pl.ds(k * 16, 16)] = w16
            plsc.addupdate_scatter(den_v, [dst16], w16)

    def scale_rows(buf, w_v):
        @pl.loop(0, G // 16)
        def _(q):
            w16 = w_v[pl.ds(q * 16, 16)]
            for u in range(16):
                e = q * 16 + u
                wv = w16[u]
                for kk in range(D // 16):
                    buf[e, pl.ds(kk * 16, 16)] = buf[e, pl.ds(kk * 16, 16)] * wv

    def drain(buf, sem):
        pltpu.make_async_copy(hp.at[pl.ds(0, G)], buf, sem).wait()

    pltpu.sync_copy(srcg.at[wid, 0], src_v)
    pltpu.sync_copy(dstg.at[wid, 0], dst_v)
    pltpu.async_copy(bufA, num_sh.at[dst_v.at[0]], semSA, add=True)
    pltpu.async_copy(bufB, num_sh.at[dst_v.at[1]], semSB, add=True)

    @pl.loop(0, NSUP)
    def _(sg):
        for p in range(GSUP // 2):
            jA, jB = 2 * p, 2 * p + 1
            drain(bufA, semSA)
            if p == 0:
                drain(bufB, semSB)
                pltpu.sync_copy(srcg.at[wid, sg], src_v)
                pltpu.sync_copy(dstg.at[wid, sg], dst_v)
            gA = pltpu.async_copy(hp.at[src_v.at[jA]], bufA, semGA)
            compute_w(jA, wA)
            gA.wait()
            if p != 0:
                drain(bufB, semSB)
            gB = pltpu.async_copy(hp.at[src_v.at[jB]], bufB, semGB)
            scale_rows(bufA, wA)
            pltpu.async_copy(bufA, num_sh.at[dst_v.at[jA]], semSA, add=True)
            compute_w(jB, wB)
            gB.wait()
            scale_rows(bufB, wB)
            pltpu.async_copy(bufB, num_sh.at[dst_v.at[jB]], semSB, add=True)

    drain(bufA, semSA)
    drain(bufB, semSB)

    pltpu.sync_copy(den_v, den_out.at[pl.ds(wid * NP, NT)])

    plsc.subcore_barrier()
    for j in range(STRIPE // G):
        pltpu.sync_copy(num_sh.at[pl.ds(base + j * G, G)], bufA)
        pltpu.sync_copy(bufA, num_out.at[c, pl.ds(base + j * G, G)])


_sc_edge = functools.partial(
    pl.kernel,
    out_type=[
        jax.ShapeDtypeStruct((SCNC, NP, D), jnp.float32),
        jax.ShapeDtypeStruct((NW * NP,), jnp.float32),
    ],
    mesh=plsc.VectorSubcoreMesh(core_axis_name="c", subcore_axis_name="s"),
    compiler_params=pltpu.CompilerParams(needs_layout_passes=False),
    scratch_types=[
        pltpu.VMEM((GSUP, G), jnp.int32),
        pltpu.VMEM((GSUP, G), jnp.int32),
        pltpu.VMEM((NT,), jnp.float32),
        pltpu.VMEM((NT,), jnp.float32),
        pltpu.VMEM((G,), jnp.float32),
        pltpu.VMEM((G,), jnp.float32),
        pltpu.VMEM((NT,), jnp.float32),
        pltpu.VMEM((G, D), jnp.float32),
        pltpu.VMEM((G, D), jnp.float32),
        pltpu.VMEM_SHARED((NP, D), jnp.float32),
        pltpu.SemaphoreType.DMA,
        pltpu.SemaphoreType.DMA,
        pltpu.SemaphoreType.DMA,
        pltpu.SemaphoreType.DMA,
    ],
)(_sc_edge_kernel)


def _tc_call(body, out_shape):
    return pl.pallas_call(body, out_shape=out_shape)


_node_arrs = [
    jax.ShapeDtypeStruct((NP, D), jnp.float32),
    jax.ShapeDtypeStruct((NP,), jnp.float32),
    jax.ShapeDtypeStruct((NP,), jnp.float32),
]


def kernel(x, adj_t, batch, W1, a_src1, a_dst1, b1, W2, a_src2, a_dst2, b2):
    xp = jnp.zeros((NP, D), jnp.float32).at[:N].set(x)
    pad = jnp.full((EP - E,), N, jnp.int32)
    srcg = jnp.concatenate([adj_t[0], pad]).reshape(NW, NSUP, GSUP, G)
    dstg = jnp.concatenate([adj_t[1], pad]).reshape(NW, NSUP, GSUP, G)

    h1, as1, ad1 = _tc_call(_tc_pro_kernel, _node_arrs)(xp, W1, a_src1, a_dst1)
    nump1, denp1 = _sc_edge(h1, as1, ad1, srcg, dstg)
    h2, as2, ad2 = _tc_call(_tc_mid_kernel, _node_arrs)(
        nump1, denp1, h1, as1, ad1, b1, W2, a_src2, a_dst2)
    nump2, denp2 = _sc_edge(h2, as2, ad2, srcg, dstg)
    out = _tc_call(_tc_fin_kernel, [
        jax.ShapeDtypeStruct((NUM_GRAPHS, D), jnp.float32),
    ])(nump2, denp2, h2, as2, ad2, b2, batch)
    return out[0]
